# baseline (speedup 1.0000x reference)
import functools

import jax
import jax.numpy as jnp
from jax import lax
from jax.experimental import pallas as pl
from jax.experimental.pallas import tpu as pltpu
from jax.experimental.pallas import tpu_sc as plsc

_SIZES = (6, 10, 2, 1, 1, 1, 1, 2, 1, 1, 2, 2)
_NF = 12
_B = 16384
_NC, _NS, _L = 2, 16, 16
_NW = _NC * _NS
_COLS = _B // _NW
_GROUPS = _COLS // _L

_mesh = plsc.VectorSubcoreMesh(core_axis_name="c", subcore_axis_name="s")


@functools.partial(
    pl.kernel,
    out_type=jax.ShapeDtypeStruct((_B,), jnp.float32),
    mesh=_mesh,
    scratch_types=[
        pltpu.VMEM((_NF, _COLS), jnp.int32),
        pltpu.VMEM((_NF, _COLS), jnp.float32),
        pltpu.VMEM((13 * _L,), jnp.float32),
        pltpu.VMEM((_COLS,), jnp.float32),
        pltpu.SemaphoreType.DMA,
    ],
    compiler_params=pltpu.CompilerParams(
        needs_layout_passes=False, use_tc_tiling_on_sc=True),
)
def _sc_fm(xt_hbm, wt_hbm, e0, e1, e2, e3, e4, e5, e6, e7, e8, e9, e10, e11,
           bias_hbm, out_hbm, x_v, w_v, tab_v, out_v, sem):
    wid = lax.axis_index("s") * _NC + lax.axis_index("c")
    col0 = wid * _COLS
    tabs = (e0, e1, e2, e3, e4, e5, e6, e7, e8, e9, e10, e11)
    copies = [
        pltpu.async_copy(xt_hbm.at[:, pl.ds(col0, _COLS)], x_v, sem),
        pltpu.async_copy(wt_hbm.at[:, pl.ds(col0, _COLS)], w_v, sem),
        pltpu.async_copy(bias_hbm, tab_v.at[pl.ds(12 * _L, 1)], sem),
    ]
    for i in range(_NF):
        copies.append(
            pltpu.async_copy(tabs[i], tab_v.at[pl.ds(i * _L, _SIZES[i])], sem)
        )
    for c in copies:
        c.wait()

    bias_vec = plsc.load_gather(tab_v, [jnp.full((_L,), 12 * _L, jnp.int32)])
    slot_refs = [tab_v.at[pl.ds(i * _L, _L)] for i in range(_NF)]

    @plsc.parallel_loop(0, _GROUPS, unroll=2)
    def _group(g):
        acc = bias_vec
        for i in range(_NF):
            xi = x_v[i, pl.ds(g * _L, _L)]
            wi = w_v[i, pl.ds(g * _L, _L)]
            tv = plsc.load_gather(slot_refs[i], [xi])
            acc = acc + wi * tv
        out_v[pl.ds(g * _L, _L)] = acc

    pltpu.sync_copy(out_v, out_hbm.at[pl.ds(col0, _COLS)])


def kernel(X, weight, emb0, emb1, emb2, emb3, emb4, emb5, emb6, emb7, emb8,
           emb9, emb10, emb11, bias):
    tabs = [t.reshape(-1) for t in
            (emb0, emb1, emb2, emb3, emb4, emb5, emb6, emb7, emb8, emb9,
             emb10, emb11)]
    return _sc_fm(X.T, weight.T, *tabs, bias)


# speedup vs baseline: 1.0223x; 1.0223x over previous
import functools

import jax
import jax.numpy as jnp
from jax import lax
from jax.experimental import pallas as pl
from jax.experimental.pallas import tpu as pltpu
from jax.experimental.pallas import tpu_sc as plsc

_SIZES = (6, 10, 2, 1, 1, 1, 1, 2, 1, 1, 2, 2)
_NF = 12
_B = 16384
_NC, _NS, _L = 2, 16, 16
_NW = _NC * _NS
_COLS = _B // _NW
_GROUPS = _COLS // _L

_mesh = plsc.VectorSubcoreMesh(core_axis_name="c", subcore_axis_name="s")


@functools.partial(
    pl.kernel,
    out_type=jax.ShapeDtypeStruct((_B,), jnp.float32),
    mesh=_mesh,
    scratch_types=[
        pltpu.VMEM((_NF, _COLS), jnp.int32),
        pltpu.VMEM((_NF, _COLS), jnp.float32),
        pltpu.VMEM((13 * _L,), jnp.float32),
        pltpu.VMEM((_COLS,), jnp.float32),
        pltpu.SemaphoreType.DMA,
    ],
    compiler_params=pltpu.CompilerParams(
        needs_layout_passes=False, use_tc_tiling_on_sc=True),
)
def _sc_fm(xt_hbm, wt_hbm, e0, e1, e2, e3, e4, e5, e6, e7, e8, e9, e10, e11,
           bias_hbm, out_hbm, x_v, w_v, tab_v, out_v, sem):
    wid = lax.axis_index("s") * _NC + lax.axis_index("c")
    col0 = wid * _COLS
    tabs = (e0, e1, e2, e3, e4, e5, e6, e7, e8, e9, e10, e11)
    copies = [
        pltpu.async_copy(xt_hbm.at[:, pl.ds(col0, _COLS)], x_v, sem),
        pltpu.async_copy(wt_hbm.at[:, pl.ds(col0, _COLS)], w_v, sem),
        pltpu.async_copy(bias_hbm, tab_v.at[pl.ds(12 * _L, 1)], sem),
    ]
    for i in range(_NF):
        copies.append(
            pltpu.async_copy(tabs[i], tab_v.at[pl.ds(i * _L, _SIZES[i])], sem)
        )
    for c in copies:
        c.wait()

    bias_vec = plsc.load_gather(tab_v, [jnp.full((_L,), 12 * _L, jnp.int32)])
    slot_refs = [tab_v.at[pl.ds(i * _L, _L)] for i in range(_NF)]

    zeros = jnp.zeros((_L,), jnp.int32)

    def entry(i, k):
        return plsc.load_gather(slot_refs[i], [zeros + k])

    t_one = {i: entry(i, 0) for i in range(_NF) if _SIZES[i] == 1}
    t_two = {i: (entry(i, 0), entry(i, 1))
             for i in range(_NF) if _SIZES[i] == 2}

    @pl.loop(0, _GROUPS)
    def _group(g):
        acc = bias_vec
        for i in range(_NF):
            wi = w_v[i, pl.ds(g * _L, _L)]
            if _SIZES[i] == 1:
                tv = t_one[i]
            elif _SIZES[i] == 2:
                xi = x_v[i, pl.ds(g * _L, _L)]
                t0, t1 = t_two[i]
                tv = jnp.where(xi == 0, t0, t1)
            else:
                xi = x_v[i, pl.ds(g * _L, _L)]
                tv = plsc.load_gather(slot_refs[i], [xi])
            acc = acc + wi * tv
        out_v[pl.ds(g * _L, _L)] = acc

    pltpu.sync_copy(out_v, out_hbm.at[pl.ds(col0, _COLS)])


def kernel(X, weight, emb0, emb1, emb2, emb3, emb4, emb5, emb6, emb7, emb8,
           emb9, emb10, emb11, bias):
    tabs = [t.reshape(-1) for t in
            (emb0, emb1, emb2, emb3, emb4, emb5, emb6, emb7, emb8, emb9,
             emb10, emb11)]
    return _sc_fm(X.T, weight.T, *tabs, bias)


# speedup vs baseline: 1.0350x; 1.0125x over previous
import functools

import jax
import jax.numpy as jnp
from jax import lax
from jax.experimental import pallas as pl
from jax.experimental.pallas import tpu as pltpu
from jax.experimental.pallas import tpu_sc as plsc

_SIZES = (6, 10, 2, 1, 1, 1, 1, 2, 1, 1, 2, 2)
_NF = 12
_B = 16384
_NC, _NS, _L = 2, 16, 16
_NW = _NC * _NS
_COLS = _B // _NW
_GROUPS = _COLS // _L

_mesh = plsc.VectorSubcoreMesh(core_axis_name="c", subcore_axis_name="s")


@functools.partial(
    pl.kernel,
    out_type=jax.ShapeDtypeStruct((_B,), jnp.float32),
    mesh=_mesh,
    scratch_types=[
        pltpu.VMEM((_NF, _COLS), jnp.int32),
        pltpu.VMEM((_NF, _COLS), jnp.float32),
        pltpu.VMEM((13 * _L,), jnp.float32),
        pltpu.VMEM((_COLS,), jnp.float32),
        pltpu.SemaphoreType.DMA,
    ],
    compiler_params=pltpu.CompilerParams(
        needs_layout_passes=False, use_tc_tiling_on_sc=True),
)
def _sc_fm(xt_hbm, wt_hbm, e0, e1, e2, e3, e4, e5, e6, e7, e8, e9, e10, e11,
           bias_hbm, out_hbm, x_v, w_v, tab_v, out_v, sem):
    wid = lax.axis_index("s") * _NC + lax.axis_index("c")
    col0 = wid * _COLS
    tabs = (e0, e1, e2, e3, e4, e5, e6, e7, e8, e9, e10, e11)
    copies = [
        pltpu.async_copy(xt_hbm.at[pl.ds(0, 3), pl.ds(col0, _COLS)],
                         x_v.at[pl.ds(0, 3)], sem),
        pltpu.async_copy(xt_hbm.at[pl.ds(7, 1), pl.ds(col0, _COLS)],
                         x_v.at[pl.ds(7, 1)], sem),
        pltpu.async_copy(xt_hbm.at[pl.ds(10, 2), pl.ds(col0, _COLS)],
                         x_v.at[pl.ds(10, 2)], sem),
        pltpu.async_copy(wt_hbm.at[:, pl.ds(col0, _COLS)], w_v, sem),
        pltpu.async_copy(bias_hbm, tab_v.at[pl.ds(12 * _L, 1)], sem),
    ]
    for i in range(_NF):
        copies.append(
            pltpu.async_copy(tabs[i], tab_v.at[pl.ds(i * _L, _SIZES[i])], sem)
        )
    for c in copies:
        c.wait()

    bias_vec = plsc.load_gather(tab_v, [jnp.full((_L,), 12 * _L, jnp.int32)])
    slot_refs = [tab_v.at[pl.ds(i * _L, _L)] for i in range(_NF)]

    zeros = jnp.zeros((_L,), jnp.int32)

    def entry(i, k):
        return plsc.load_gather(slot_refs[i], [zeros + k])

    t_one = {i: entry(i, 0) for i in range(_NF) if _SIZES[i] == 1}
    t_two = {i: (entry(i, 0), entry(i, 1))
             for i in range(_NF) if _SIZES[i] == 2}

    @pl.loop(0, _GROUPS)
    def _group(g):
        acc = bias_vec
        for i in range(_NF):
            wi = w_v[i, pl.ds(g * _L, _L)]
            if _SIZES[i] == 1:
                tv = t_one[i]
            elif _SIZES[i] == 2:
                xi = x_v[i, pl.ds(g * _L, _L)]
                t0, t1 = t_two[i]
                tv = jnp.where(xi == 0, t0, t1)
            else:
                xi = x_v[i, pl.ds(g * _L, _L)]
                tv = plsc.load_gather(slot_refs[i], [xi])
            acc = acc + wi * tv
        out_v[pl.ds(g * _L, _L)] = acc

    pltpu.sync_copy(out_v, out_hbm.at[pl.ds(col0, _COLS)])


def kernel(X, weight, emb0, emb1, emb2, emb3, emb4, emb5, emb6, emb7, emb8,
           emb9, emb10, emb11, bias):
    tabs = [t.reshape(-1) for t in
            (emb0, emb1, emb2, emb3, emb4, emb5, emb6, emb7, emb8, emb9,
             emb10, emb11)]
    return _sc_fm(X.T, weight.T, *tabs, bias)
